# baseline (speedup 1.0000x reference)
import dataclasses
import functools

import jax
import jax.numpy as jnp
from jax import lax
from jax.experimental import pallas as pl
from jax.experimental.pallas import tpu as pltpu
from jax.experimental.pallas import tpu_sc as plsc

NUM = 1000000
DIM = 32
BATCH = 16384

NC = 2
NS = 16
L = 16
NW = NC * NS
BPW = BATCH // NW
CHUNK = 128
NCHUNK = BPW // CHUNK


def _compiler_params():
    cp = pltpu.CompilerParams()
    fields = pltpu.CompilerParams.__dataclass_fields__
    if "needs_layout_passes" in fields:
        cp = dataclasses.replace(cp, needs_layout_passes=False)
    if "use_tc_tiling_on_sc" in fields:
        cp = dataclasses.replace(cp, use_tc_tiling_on_sc=False)
    return cp


def _dot_kernel(aidx_hbm, bidx_hbm, a_hbm, b_hbm, out_hbm,
                ai_v, bi_v, ar_v, br_v, o_v, sem_a, sem_b):
    wid = lax.axis_index("s") * NC + lax.axis_index("c")
    base = wid * BPW

    pltpu.sync_copy(aidx_hbm.at[wid], ai_v)
    pltpu.sync_copy(bidx_hbm.at[wid], bi_v)

    copies = []
    for j in range(NCHUNK):
        dst = pl.ds(j * CHUNK, CHUNK)
        copies.append(pltpu.async_copy(a_hbm.at[ai_v.at[j]], ar_v.at[dst], sem_a))
        copies.append(pltpu.async_copy(b_hbm.at[bi_v.at[j]], br_v.at[dst], sem_b))

    lane = lax.iota(jnp.int32, 16)

    for j in range(NCHUNK):
        copies[2 * j].wait()
        copies[2 * j + 1].wait()

        @pl.loop(j * CHUNK, (j + 1) * CHUNK, step=L)
        def _(g):
            rows = lane + g
            acc = None
            for d in range(DIM):
                col = jnp.full((L,), d, jnp.int32)
                av = plsc.load_gather(ar_v, [rows, col])
                bv = plsc.load_gather(br_v, [rows, col])
                prod = av * bv
                acc = prod if acc is None else acc + prod
            o_v[pl.ds(g, L)] = acc

    pltpu.sync_copy(o_v, out_hbm.at[pl.ds(base, BPW)])


@jax.jit
def kernel(aIdx, bIdx, A, B):
    aIdx = aIdx.reshape(NW, NCHUNK, CHUNK).astype(jnp.int32)
    bIdx = bIdx.reshape(NW, NCHUNK, CHUNK).astype(jnp.int32)
    mesh = plsc.VectorSubcoreMesh(core_axis_name="c", subcore_axis_name="s")
    run = functools.partial(
        pl.kernel,
        mesh=mesh,
        out_type=jax.ShapeDtypeStruct((BATCH,), jnp.float32),
        scratch_types=[
            pltpu.VMEM((NCHUNK, CHUNK), jnp.int32),
            pltpu.VMEM((NCHUNK, CHUNK), jnp.int32),
            pltpu.VMEM((BPW, DIM), jnp.float32),
            pltpu.VMEM((BPW, DIM), jnp.float32),
            pltpu.VMEM((BPW,), jnp.float32),
            pltpu.SemaphoreType.DMA,
            pltpu.SemaphoreType.DMA,
        ],
        compiler_params=_compiler_params(),
    )(_dot_kernel)
    return run(aIdx, bIdx, A, B)
